# baseline (speedup 1.0000x reference)
import functools

import jax
import jax.numpy as jnp
from jax.experimental import pallas as pl
from jax.experimental.pallas import tpu as pltpu


def _ca_kernel(x_ref, w1t_ref, b1_ref, w2t_ref, b2_ref, o_ref, *, inv_hw):
    x = x_ref[...]
    y = jnp.sum(x, axis=1, keepdims=True, dtype=jnp.float32) * inv_hw
    h = jnp.maximum(
        jnp.dot(w1t_ref[...], y, preferred_element_type=jnp.float32)
        + b1_ref[...], 0.0)
    s = jax.nn.sigmoid(
        jnp.dot(w2t_ref[...], h, preferred_element_type=jnp.float32)
        + b2_ref[...])
    o_ref[...] = (x * s.astype(x.dtype)).astype(o_ref.dtype)


def kernel(x, w1, b1, w2, b2):
    N, C, H, W = x.shape
    HW = H * W
    Cr = w1.shape[1]
    inv_hw = float(1.0 / HW)
    itemsize = jnp.dtype(x.dtype).itemsize

    w1t = w1.T
    b1c = b1.reshape(Cr, 1)
    w2t = w2.T
    b2c = b2.reshape(C, 1)

    x2 = x.reshape(N * C, HW)

    cost = pl.CostEstimate(
        flops=2 * N * C * HW + 4 * N * C * Cr,
        transcendentals=N * C,
        bytes_accessed=2 * N * C * HW * itemsize,
    )

    out = pl.pallas_call(
        functools.partial(_ca_kernel, inv_hw=inv_hw),
        out_shape=jax.ShapeDtypeStruct((N * C, HW), x.dtype),
        grid=(N,),
        in_specs=[
            pl.BlockSpec((C, HW), lambda n: (n, 0)),
            pl.BlockSpec((Cr, C), lambda n: (0, 0)),
            pl.BlockSpec((Cr, 1), lambda n: (0, 0)),
            pl.BlockSpec((C, Cr), lambda n: (0, 0)),
            pl.BlockSpec((C, 1), lambda n: (0, 0)),
        ],
        out_specs=pl.BlockSpec((C, HW), lambda n: (n, 0)),
        compiler_params=pltpu.CompilerParams(
            dimension_semantics=("parallel",),
            vmem_limit_bytes=48 << 20),
        cost_estimate=cost,
    )(x2, w1t, b1c, w2t, b2c)
    return out.reshape(N, C, H, W)


# speedup vs baseline: 3.5323x; 3.5323x over previous
import functools

import jax
import jax.numpy as jnp
from jax.experimental import pallas as pl
from jax.experimental.pallas import tpu as pltpu


def _ca_kernel(x_ref, w1_ref, b1_ref, w2_ref, b2_ref, o_ref, *, inv_hw):
    x = x_ref[...]
    y = jnp.sum(x, axis=(2, 3), dtype=jnp.float32) * inv_hw
    h = jnp.maximum(
        jnp.dot(y, w1_ref[...], preferred_element_type=jnp.float32)
        + b1_ref[...], 0.0)
    s = jax.nn.sigmoid(
        jnp.dot(h, w2_ref[...], preferred_element_type=jnp.float32)
        + b2_ref[...])
    o_ref[...] = (x * s.astype(x.dtype)[:, :, None, None]).astype(o_ref.dtype)


def kernel(x, w1, b1, w2, b2):
    N, C, H, W = x.shape
    HW = H * W
    Cr = w1.shape[1]
    inv_hw = float(1.0 / HW)
    itemsize = jnp.dtype(x.dtype).itemsize

    cost = pl.CostEstimate(
        flops=2 * N * C * HW + 4 * N * C * Cr,
        transcendentals=N * C,
        bytes_accessed=2 * N * C * HW * itemsize,
    )

    return pl.pallas_call(
        functools.partial(_ca_kernel, inv_hw=inv_hw),
        out_shape=jax.ShapeDtypeStruct((N, C, H, W), x.dtype),
        grid=(N,),
        in_specs=[
            pl.BlockSpec((1, C, H, W), lambda n: (n, 0, 0, 0)),
            pl.BlockSpec((C, Cr), lambda n: (0, 0)),
            pl.BlockSpec((1, Cr), lambda n: (0, 0)),
            pl.BlockSpec((Cr, C), lambda n: (0, 0)),
            pl.BlockSpec((1, C), lambda n: (0, 0)),
        ],
        out_specs=pl.BlockSpec((1, C, H, W), lambda n: (n, 0, 0, 0)),
        compiler_params=pltpu.CompilerParams(
            dimension_semantics=("parallel",),
            vmem_limit_bytes=57 << 20),
        cost_estimate=cost,
    )(x, w1, b1, w2, b2)
